# Initial kernel scaffold; baseline (speedup 1.0000x reference)
import jax
import jax.numpy as jnp
from jax import lax
from jax.experimental import pallas as pl
from jax.experimental.pallas import tpu as pltpu
from jax.experimental.pallas import tpu_sc as plsc

N_NODES = 10000
N_EDGES = 160000
D_FEAT = 256
DH = D_FEAT // 2
NS = 16
EPT = N_EDGES // NS
CHUNK = 80
NCHUNK = EPT // CHUNK
RPT = N_NODES // NS
RSUB = 125
NRSUB = RPT // RSUB


def _body(x0, x1, srcr, dstr, wr, out0, out1,
          src_v, dst_v, w_v, rows_v, zb, zs, out_sh, rs_sh, sem):
    c = lax.axis_index("c")
    s = lax.axis_index("s")

    zero16 = jnp.zeros((16,), jnp.float32)

    def zrow(i, _):
        for j in range(8):
            zb[i, pl.ds(16 * j, 16)] = zero16
        return 0
    lax.fori_loop(0, RSUB, zrow, 0)

    def zrs(i, _):
        zs[pl.ds(i * 16, 16)] = zero16
        return 0
    lax.fori_loop(0, 40, zrs, 0)

    row0 = s * RPT
    for k in range(NRSUB):
        pltpu.sync_copy(zb, out_sh.at[pl.ds(row0 + k * RSUB, RSUB), :])
    pltpu.sync_copy(zs, rs_sh.at[pl.ds(s * 640, 640)])
    plsc.subcore_barrier()

    ebase = s * EPT

    def chunk_body(i, _):
        b = ebase + i * CHUNK
        pltpu.sync_copy(srcr.at[pl.ds(b, CHUNK)], src_v)
        pltpu.sync_copy(dstr.at[pl.ds(b, CHUNK)], dst_v)
        pltpu.sync_copy(wr.at[pl.ds(b, CHUNK)], w_v)

        pltpu.sync_copy(w_v, rs_sh.at[dst_v], add=True)

        @pl.when(c == 0)
        def _():
            pltpu.async_copy(x0.at[src_v], rows_v, sem).wait()

        @pl.when(c == 1)
        def _():
            pltpu.async_copy(x1.at[src_v], rows_v, sem).wait()

        def erow(e, _):
            ws = w_v[e]
            for j in range(8):
                sl = pl.ds(16 * j, 16)
                rows_v[e, sl] = rows_v[e, sl] * ws
            return 0
        lax.fori_loop(0, CHUNK, erow, 0)

        pltpu.sync_copy(rows_v, out_sh.at[dst_v], add=True)
        return 0

    lax.fori_loop(0, NCHUNK, chunk_body, 0)
    plsc.subcore_barrier()

    abase = (row0 // 8) * 8
    off = row0 - abase
    pltpu.sync_copy(rs_sh.at[pl.ds(abase, 640)], zs)

    def inv_chunk(i, _):
        sl = pl.ds(i * 16, 16)
        zs[sl] = 1.0 / jnp.maximum(zs[sl], 1e-12)
        return 0
    lax.fori_loop(0, 40, inv_chunk, 0)

    for k in range(NRSUB):
        r0 = row0 + k * RSUB
        pltpu.sync_copy(out_sh.at[pl.ds(r0, RSUB), :], zb)

        def nrow(r, _):
            iv = zs[off + k * RSUB + r]
            for j in range(8):
                sl = pl.ds(16 * j, 16)
                zb[r, sl] = zb[r, sl] * iv
            return 0
        lax.fori_loop(0, RSUB, nrow, 0)

        @pl.when(c == 0)
        def _():
            pltpu.sync_copy(zb, out0.at[pl.ds(r0, RSUB), :])

        @pl.when(c == 1)
        def _():
            pltpu.sync_copy(zb, out1.at[pl.ds(r0, RSUB), :])


def _make_kernel():
    mesh = plsc.VectorSubcoreMesh(core_axis_name="c", subcore_axis_name="s")
    return pl.kernel(
        _body,
        out_type=(
            jax.ShapeDtypeStruct((N_NODES, DH), jnp.float32),
            jax.ShapeDtypeStruct((N_NODES, DH), jnp.float32),
        ),
        mesh=mesh,
        scratch_types=[
            pltpu.VMEM((CHUNK,), jnp.int32),
            pltpu.VMEM((CHUNK,), jnp.int32),
            pltpu.VMEM((CHUNK,), jnp.float32),
            pltpu.VMEM((CHUNK, DH), jnp.float32),
            pltpu.VMEM((RSUB, DH), jnp.float32),
            pltpu.VMEM((640,), jnp.float32),
            pltpu.VMEM_SHARED((N_NODES, DH), jnp.float32),
            pltpu.VMEM_SHARED((N_NODES + 240,), jnp.float32),
            pltpu.SemaphoreType.DMA,
        ],
    )


@jax.jit
def kernel(x, edge_index, edge_weight):
    x0 = x[:, :DH]
    x1 = x[:, DH:]
    src = edge_index[0]
    dst = edge_index[1]
    out0, out1 = _make_kernel()(x0, x1, src, dst, edge_weight)
    return jnp.concatenate([out0, out1], axis=1)


# speedup vs baseline: 4.5748x; 4.5748x over previous
import jax
import jax.numpy as jnp
from jax import lax
from jax.experimental import pallas as pl
from jax.experimental.pallas import tpu as pltpu
from jax.experimental.pallas import tpu_sc as plsc

N_NODES = 10000
N_EDGES = 160000
D_FEAT = 256
DH = D_FEAT // 2
NS = 16
EPT = N_EDGES // NS
CHUNK = 80
NCHUNK = EPT // CHUNK
N_PAD = 10240
RPT = N_PAD // NS
RSUB = 128
NRSUB = RPT // RSUB


def _body(x0, x1, srcr, dstr, wr, out0, out1,
          src_v, dst_v, w_v, rows_v, zb, zs, out_sh, rs_sh, sem):
    c = lax.axis_index("c")
    s = lax.axis_index("s")

    zero16 = jnp.zeros((16,), jnp.float32)

    def zrow(i, _):
        for j in range(8):
            zb[i, pl.ds(16 * j, 16)] = zero16
        return 0
    lax.fori_loop(0, RSUB, zrow, 0)

    def zrs(i, _):
        zs[pl.ds(i * 16, 16)] = zero16
        return 0
    lax.fori_loop(0, RPT // 16, zrs, 0)

    row0 = s * RPT
    for k in range(NRSUB):
        pltpu.sync_copy(zb, out_sh.at[pl.ds(row0 + k * RSUB, RSUB), :])
    pltpu.sync_copy(zs, rs_sh.at[pl.ds(row0, RPT)])
    plsc.subcore_barrier()

    ebase = s * EPT

    def chunk_body(i, _):
        b = ebase + i * CHUNK
        pltpu.sync_copy(srcr.at[pl.ds(b, CHUNK)], src_v)
        pltpu.sync_copy(dstr.at[pl.ds(b, CHUNK)], dst_v)
        pltpu.sync_copy(wr.at[pl.ds(b, CHUNK)], w_v)

        pltpu.sync_copy(w_v, rs_sh.at[dst_v], add=True)

        @pl.when(c == 0)
        def _():
            pltpu.async_copy(x0.at[src_v], rows_v, sem).wait()

        @pl.when(c == 1)
        def _():
            pltpu.async_copy(x1.at[src_v], rows_v, sem).wait()

        def egrp(g, _):
            wv = w_v[pl.ds(g * 16, 16)]
            for e in range(16):
                ws = wv[e]
                row = g * 16 + e
                for j in range(8):
                    sl = pl.ds(16 * j, 16)
                    rows_v[row, sl] = rows_v[row, sl] * ws
            return 0
        lax.fori_loop(0, CHUNK // 16, egrp, 0)

        pltpu.sync_copy(rows_v, out_sh.at[dst_v], add=True)
        return 0

    lax.fori_loop(0, NCHUNK, chunk_body, 0)
    plsc.subcore_barrier()

    pltpu.sync_copy(rs_sh.at[pl.ds(row0, RPT)], zs)

    def inv_chunk(i, _):
        sl = pl.ds(i * 16, 16)
        zs[sl] = 1.0 / jnp.maximum(zs[sl], 1e-12)
        return 0
    lax.fori_loop(0, RPT // 16, inv_chunk, 0)

    for k in range(NRSUB):
        r0 = row0 + k * RSUB
        pltpu.sync_copy(out_sh.at[pl.ds(r0, RSUB), :], zb)

        def ngrp(g, _):
            ivv = zs[pl.ds(k * RSUB + g * 16, 16)]
            for e in range(16):
                iv = ivv[e]
                row = g * 16 + e
                for j in range(8):
                    sl = pl.ds(16 * j, 16)
                    zb[row, sl] = zb[row, sl] * iv
            return 0
        lax.fori_loop(0, RSUB // 16, ngrp, 0)

        def wout(ref, nrows):
            @pl.when(c == 0)
            def _():
                pltpu.sync_copy(zb.at[pl.ds(0, nrows), :] if nrows != RSUB else zb,
                                out0.at[pl.ds(r0, nrows), :])

            @pl.when(c == 1)
            def _():
                pltpu.sync_copy(zb.at[pl.ds(0, nrows), :] if nrows != RSUB else zb,
                                out1.at[pl.ds(r0, nrows), :])

        if k < 3:
            wout(zb, RSUB)
        elif k == 3:
            @pl.when(s < 15)
            def _():
                wout(zb, RSUB)

            @pl.when(s == 15)
            def _():
                wout(zb, 16)
        else:
            @pl.when(s < 15)
            def _():
                wout(zb, RSUB)


def _make_kernel():
    mesh = plsc.VectorSubcoreMesh(core_axis_name="c", subcore_axis_name="s")
    return pl.kernel(
        _body,
        out_type=(
            jax.ShapeDtypeStruct((N_NODES, DH), jnp.float32),
            jax.ShapeDtypeStruct((N_NODES, DH), jnp.float32),
        ),
        mesh=mesh,
        scratch_types=[
            pltpu.VMEM((CHUNK,), jnp.int32),
            pltpu.VMEM((CHUNK,), jnp.int32),
            pltpu.VMEM((CHUNK,), jnp.float32),
            pltpu.VMEM((CHUNK, DH), jnp.float32),
            pltpu.VMEM((RSUB, DH), jnp.float32),
            pltpu.VMEM((RPT,), jnp.float32),
            pltpu.VMEM_SHARED((N_PAD, DH), jnp.float32),
            pltpu.VMEM_SHARED((N_PAD,), jnp.float32),
            pltpu.SemaphoreType.DMA,
        ],
    )


@jax.jit
def kernel(x, edge_index, edge_weight):
    x0 = x[:, :DH]
    x1 = x[:, DH:]
    src = edge_index[0]
    dst = edge_index[1]
    out0, out1 = _make_kernel()(x0, x1, src, dst, edge_weight)
    return jnp.concatenate([out0, out1], axis=1)


# speedup vs baseline: 7.9386x; 1.7353x over previous
import jax
import jax.numpy as jnp
from jax import lax
from jax.experimental import pallas as pl
from jax.experimental.pallas import tpu as pltpu
from jax.experimental.pallas import tpu_sc as plsc

N_NODES = 10000
N_EDGES = 160000
D_FEAT = 256
QW = 64
NS = 16
EPT = N_EDGES // NS
CHUNK = 80
NCHUNK = EPT // CHUNK
N_PAD = 10240
RPT = N_PAD // NS
BLK = 80
NBLK = RPT // BLK
NBUF = 4
LOOKAHEAD = 2
RS_LAG = 8


def _body(xt, src2, dst2, w2, o00, o01, o10, o11,
          src_v, dst_v, w_v, rb0, rb1, rb2, rb3, zbuf, rsb,
          out_sh, rs_sh, gs0, gs1, gs2, gs3, ss0, ss1, ss2, ss3, rssem):
    c = lax.axis_index("c")
    s = lax.axis_index("s")
    rows = [rb0, rb1, rb2, rb3]
    gsems = [gs0, gs1, gs2, gs3]
    ssems = [ss0, ss1, ss2, ss3]

    zero16 = jnp.zeros((16,), jnp.float32)

    pltpu.sync_copy(src2.at[s], src_v)
    pltpu.sync_copy(dst2.at[s], dst_v)
    pltpu.sync_copy(w2.at[s], w_v)

    def add_off(i, _):
        for g in range(CHUNK // 16):
            sl = pl.ds(g * 16, 16)
            src_v[i, sl] = src_v[i, sl] * 4 + c * 2
        return 0
    lax.fori_loop(0, NCHUNK, add_off, 0)

    def zrow(i, _):
        for j in range(QW // 16):
            zbuf[i, pl.ds(16 * j, 16)] = zero16
        return 0
    lax.fori_loop(0, BLK, zrow, 0)

    def zrs(i, _):
        rsb[pl.ds(i * 16, 16)] = zero16
        return 0
    lax.fori_loop(0, RPT // 16, zrs, 0)

    row0 = s * RPT
    for k in range(NBLK):
        pltpu.sync_copy(zbuf, out_sh.at[pl.ds(row0 + k * BLK, BLK), :])
    pltpu.sync_copy(rsb, rs_sh.at[pl.ds(row0, RPT)])
    plsc.subcore_barrier()

    def scale(rb, j):
        def grp(g, _):
            wv = w_v[j, pl.ds(g * 16, 16)]
            for e in range(16):
                ws = wv[e]
                r = g * 16 + e
                for cj in range(QW // 16):
                    sl = pl.ds(16 * cj, 16)
                    rb[r, sl] = rb[r, sl] * ws
            return 0
        lax.fori_loop(0, CHUNK // 16, grp, 0)

    def issue_gather(j, b):
        pltpu.async_copy(xt.at[src_v.at[j]], rows[b], gsems[b])

    def wait_gather(j, b):
        pltpu.make_async_copy(xt.at[src_v.at[j]], rows[b], gsems[b]).wait()

    def issue_scatter(j, b):
        pltpu.async_copy(rows[b], out_sh.at[dst_v.at[j]], ssems[b], add=True)

    def wait_scatter(j, b):
        pltpu.make_async_copy(rows[b], out_sh.at[dst_v.at[j]], ssems[b]).wait()

    def issue_rs(j):
        pltpu.async_copy(w_v.at[j], rs_sh.at[dst_v.at[j]], rssem, add=True)

    def wait_rs():
        pltpu.make_async_copy(w_v.at[0], rs_sh.at[dst_v.at[0]], rssem).wait()

    def one_pass(q, _):
        p0 = q == 0

        issue_gather(0, 0)
        issue_gather(1, 1)

        def step(t, _):
            for b in range(NBUF):
                j = NBUF * t + b
                wait_gather(j, b)
                scale(rows[b], j)
                issue_scatter(j, b)

                @pl.when(p0)
                def _():
                    issue_rs(j)

                    @pl.when(j >= RS_LAG)
                    def _():
                        wait_rs()

                bn = (b + LOOKAHEAD) % NBUF

                @pl.when(j >= LOOKAHEAD)
                def _():
                    wait_scatter(j - LOOKAHEAD, bn)

                @pl.when(j + LOOKAHEAD <= NCHUNK - 1)
                def _():
                    issue_gather(j + LOOKAHEAD, bn)
            return 0

        lax.fori_loop(0, (NCHUNK - 1) // NBUF, step, 0)

        jl = NCHUNK - 1
        wait_gather(jl, 0)
        scale(rows[0], jl)
        issue_scatter(jl, 0)
        wait_scatter(jl - 2, 2)
        wait_scatter(jl - 1, 3)
        wait_scatter(jl, 0)

        @pl.when(p0)
        def _():
            issue_rs(jl)

            def drain_rs(i, _):
                wait_rs()
                return 0
            lax.fori_loop(0, RS_LAG + 1, drain_rs, 0)

        plsc.subcore_barrier()

        @pl.when(p0)
        def _():
            pltpu.sync_copy(rs_sh.at[pl.ds(row0, RPT)], rsb)

            def inv_chunk(i, _):
                sl = pl.ds(i * 16, 16)
                rsb[sl] = 1.0 / jnp.maximum(rsb[sl], 1e-12)
                return 0
            lax.fori_loop(0, RPT // 16, inv_chunk, 0)

        stage = rows[0]
        for k in range(NBLK):
            r0 = row0 + k * BLK
            pltpu.sync_copy(out_sh.at[pl.ds(r0, BLK), :], stage)

            def ngrp(g, _):
                ivv = rsb[pl.ds(k * BLK + g * 16, 16)]
                for e in range(16):
                    ive = ivv[e]
                    r = g * 16 + e
                    for cj in range(QW // 16):
                        sl = pl.ds(16 * cj, 16)
                        stage[r, sl] = stage[r, sl] * ive
                return 0
            lax.fori_loop(0, BLK // 16, ngrp, 0)

            @pl.when(jnp.logical_or(s < NS - 1, k < 5))
            def _():
                for cc, qq, ref in ((0, 0, o00), (0, 1, o01),
                                    (1, 0, o10), (1, 1, o11)):
                    @pl.when(jnp.logical_and(c == cc, q == qq))
                    def _():
                        pltpu.sync_copy(stage, ref.at[pl.ds(r0, BLK), :])

            pltpu.sync_copy(zbuf, out_sh.at[pl.ds(r0, BLK), :])

        def bump(i, _):
            for g in range(CHUNK // 16):
                sl = pl.ds(g * 16, 16)
                src_v[i, sl] = src_v[i, sl] + 1
            return 0
        lax.fori_loop(0, NCHUNK, bump, 0)
        plsc.subcore_barrier()
        return 0

    lax.fori_loop(0, 2, one_pass, 0)


def _make_kernel():
    mesh = plsc.VectorSubcoreMesh(core_axis_name="c", subcore_axis_name="s")
    row_buf = pltpu.VMEM((CHUNK, QW), jnp.float32)
    oshape = jax.ShapeDtypeStruct((N_NODES, QW), jnp.float32)
    return pl.kernel(
        _body,
        out_type=(oshape, oshape, oshape, oshape),
        mesh=mesh,
        compiler_params=pltpu.CompilerParams(use_tc_tiling_on_sc=False),
        scratch_types=[
            pltpu.VMEM((NCHUNK, CHUNK), jnp.int32),
            pltpu.VMEM((NCHUNK, CHUNK), jnp.int32),
            pltpu.VMEM((NCHUNK, CHUNK), jnp.float32),
            row_buf, row_buf, row_buf, row_buf,
            pltpu.VMEM((BLK, QW), jnp.float32),
            pltpu.VMEM((RPT,), jnp.float32),
            pltpu.VMEM_SHARED((N_PAD, QW), jnp.float32),
            pltpu.VMEM_SHARED((N_PAD,), jnp.float32),
            pltpu.SemaphoreType.DMA, pltpu.SemaphoreType.DMA,
            pltpu.SemaphoreType.DMA, pltpu.SemaphoreType.DMA,
            pltpu.SemaphoreType.DMA, pltpu.SemaphoreType.DMA,
            pltpu.SemaphoreType.DMA, pltpu.SemaphoreType.DMA,
            pltpu.SemaphoreType.DMA,
        ],
    )


@jax.jit
def kernel(x, edge_index, edge_weight):
    xt = x.reshape(4 * N_NODES, QW)
    src2 = edge_index[0].reshape(NS, NCHUNK, CHUNK)
    dst2 = edge_index[1].reshape(NS, NCHUNK, CHUNK)
    w2 = edge_weight.reshape(NS, NCHUNK, CHUNK)
    o00, o01, o10, o11 = _make_kernel()(xt, src2, dst2, w2)
    return jnp.concatenate([o00, o01, o10, o11], axis=1)


# speedup vs baseline: 8.7494x; 1.1021x over previous
import jax
import jax.numpy as jnp
from jax import lax
from jax.experimental import pallas as pl
from jax.experimental.pallas import tpu as pltpu
from jax.experimental.pallas import tpu_sc as plsc

N_NODES = 10000
N_EDGES = 160000
D_FEAT = 256
QW = 64
NS = 16
CHUNK = 128
NCH_TOT = N_EDGES // CHUNK
NCH_LO = NCH_TOT // NS
NCH_HI = NCH_LO + 1
N_EXTRA = NCH_TOT - NS * NCH_LO
N_PAD = 10240
RPT = N_PAD // NS
BLK = 80
NBLK = RPT // BLK
NBUF = 4
LOOKAHEAD = 2
RS_LAG = 8


def _body(xt, src2, dst2, w2, o00, o01, o10, o11,
          src_v, dst_v, w_v, rb0, rb1, rb2, rb3, zbuf, rsb,
          out_sh, rs_sh, gs0, gs1, gs2, gs3, ss0, ss1, ss2, ss3, rssem):
    c = lax.axis_index("c")
    s = lax.axis_index("s")
    rows = [rb0, rb1, rb2, rb3]
    gsems = [gs0, gs1, gs2, gs3]
    ssems = [ss0, ss1, ss2, ss3]

    zero16 = jnp.zeros((16,), jnp.float32)

    cb = s * NCH_LO + jnp.minimum(s, N_EXTRA)
    nc = jnp.where(s < N_EXTRA, NCH_HI, NCH_LO)
    extra = s < N_EXTRA

    pltpu.sync_copy(src2.at[pl.ds(cb, NCH_LO), :], src_v.at[pl.ds(0, NCH_LO), :])
    pltpu.sync_copy(dst2.at[pl.ds(cb, NCH_LO), :], dst_v.at[pl.ds(0, NCH_LO), :])
    pltpu.sync_copy(w2.at[pl.ds(cb, NCH_LO), :], w_v.at[pl.ds(0, NCH_LO), :])

    @pl.when(extra)
    def _():
        pltpu.sync_copy(src2.at[pl.ds(cb + NCH_LO, 1), :],
                        src_v.at[pl.ds(NCH_LO, 1), :])
        pltpu.sync_copy(dst2.at[pl.ds(cb + NCH_LO, 1), :],
                        dst_v.at[pl.ds(NCH_LO, 1), :])
        pltpu.sync_copy(w2.at[pl.ds(cb + NCH_LO, 1), :],
                        w_v.at[pl.ds(NCH_LO, 1), :])

    def add_off(i, _):
        for g in range(CHUNK // 16):
            sl = pl.ds(g * 16, 16)
            src_v[i, sl] = src_v[i, sl] * 4 + c * 2
        return 0
    lax.fori_loop(0, NCH_HI, add_off, 0)

    def zrow(i, _):
        for j in range(QW // 16):
            zbuf[i, pl.ds(16 * j, 16)] = zero16
        return 0
    lax.fori_loop(0, BLK, zrow, 0)

    def zrs(i, _):
        rsb[pl.ds(i * 16, 16)] = zero16
        return 0
    lax.fori_loop(0, RPT // 16, zrs, 0)

    row0 = s * RPT
    for k in range(NBLK):
        pltpu.sync_copy(zbuf, out_sh.at[pl.ds(row0 + k * BLK, BLK), :])
    pltpu.sync_copy(rsb, rs_sh.at[pl.ds(row0, RPT)])
    plsc.subcore_barrier()

    def scale(rb, j):
        def grp(g, _):
            wv = w_v[j, pl.ds(g * 16, 16)]
            for e in range(16):
                ws = wv[e]
                r = g * 16 + e
                for cj in range(QW // 16):
                    sl = pl.ds(16 * cj, 16)
                    rb[r, sl] = rb[r, sl] * ws
            return 0
        lax.fori_loop(0, CHUNK // 16, grp, 0)

    def issue_gather(j, b):
        pltpu.async_copy(xt.at[src_v.at[j]], rows[b], gsems[b])

    def wait_gather(j, b):
        pltpu.make_async_copy(xt.at[src_v.at[j]], rows[b], gsems[b]).wait()

    def issue_scatter(j, b):
        pltpu.async_copy(rows[b], out_sh.at[dst_v.at[j]], ssems[b], add=True)

    def wait_scatter(j, b):
        pltpu.make_async_copy(rows[b], out_sh.at[dst_v.at[j]], ssems[b]).wait()

    def issue_rs(j):
        pltpu.async_copy(w_v.at[j], rs_sh.at[dst_v.at[j]], rssem, add=True)

    def wait_rs():
        pltpu.make_async_copy(w_v.at[0], rs_sh.at[dst_v.at[0]], rssem).wait()

    def one_pass(q, _):
        p0 = q == 0

        issue_gather(0, 0)
        issue_gather(1, 1)

        def step(t, _):
            for b in range(NBUF):
                j = NBUF * t + b

                @pl.when(j < nc)
                def _():
                    wait_gather(j, b)
                    scale(rows[b], j)
                    issue_scatter(j, b)

                    @pl.when(p0)
                    def _():
                        issue_rs(j)

                        @pl.when(j >= RS_LAG)
                        def _():
                            wait_rs()

                bn = (b + LOOKAHEAD) % NBUF

                @pl.when(jnp.logical_and(j >= LOOKAHEAD, j - LOOKAHEAD < nc))
                def _():
                    wait_scatter(j - LOOKAHEAD, bn)

                @pl.when(j + LOOKAHEAD < nc)
                def _():
                    issue_gather(j + LOOKAHEAD, bn)
            return 0

        lax.fori_loop(0, NCH_HI // NBUF + 1, step, 0)

        @pl.when(extra)
        def _():
            wait_scatter(NCH_HI - 1, (NCH_HI - 1) % NBUF)

        @pl.when(p0)
        def _():
            def drain_rs(i, _):
                wait_rs()
                return 0
            lax.fori_loop(0, RS_LAG, drain_rs, 0)

        plsc.subcore_barrier()

        @pl.when(p0)
        def _():
            pltpu.sync_copy(rs_sh.at[pl.ds(row0, RPT)], rsb)

            def inv_chunk(i, _):
                sl = pl.ds(i * 16, 16)
                rsb[sl] = 1.0 / jnp.maximum(rsb[sl], 1e-12)
                return 0
            lax.fori_loop(0, RPT // 16, inv_chunk, 0)

        stage = rows[0]
        stage_blk = stage.at[pl.ds(0, BLK), :]
        for k in range(NBLK):
            r0 = row0 + k * BLK
            pltpu.sync_copy(out_sh.at[pl.ds(r0, BLK), :], stage_blk)

            def ngrp(g, _):
                ivv = rsb[pl.ds(k * BLK + g * 16, 16)]
                for e in range(16):
                    ive = ivv[e]
                    r = g * 16 + e
                    for cj in range(QW // 16):
                        sl = pl.ds(16 * cj, 16)
                        stage[r, sl] = stage[r, sl] * ive
                return 0
            lax.fori_loop(0, BLK // 16, ngrp, 0)

            @pl.when(jnp.logical_or(s < NS - 1, k < 5))
            def _():
                for cc, qq, ref in ((0, 0, o00), (0, 1, o01),
                                    (1, 0, o10), (1, 1, o11)):
                    @pl.when(jnp.logical_and(c == cc, q == qq))
                    def _():
                        pltpu.sync_copy(stage_blk, ref.at[pl.ds(r0, BLK), :])

            pltpu.sync_copy(zbuf, out_sh.at[pl.ds(r0, BLK), :])

        def bump(i, _):
            for g in range(CHUNK // 16):
                sl = pl.ds(g * 16, 16)
                src_v[i, sl] = src_v[i, sl] + 1
            return 0
        lax.fori_loop(0, NCH_HI, bump, 0)
        plsc.subcore_barrier()
        return 0

    lax.fori_loop(0, 2, one_pass, 0)


def _make_kernel():
    mesh = plsc.VectorSubcoreMesh(core_axis_name="c", subcore_axis_name="s")
    row_buf = pltpu.VMEM((CHUNK, QW), jnp.float32)
    oshape = jax.ShapeDtypeStruct((N_NODES, QW), jnp.float32)
    return pl.kernel(
        _body,
        out_type=(oshape, oshape, oshape, oshape),
        mesh=mesh,
        compiler_params=pltpu.CompilerParams(use_tc_tiling_on_sc=False),
        scratch_types=[
            pltpu.VMEM((NCH_HI, CHUNK), jnp.int32),
            pltpu.VMEM((NCH_HI, CHUNK), jnp.int32),
            pltpu.VMEM((NCH_HI, CHUNK), jnp.float32),
            row_buf, row_buf, row_buf, row_buf,
            pltpu.VMEM((BLK, QW), jnp.float32),
            pltpu.VMEM((RPT,), jnp.float32),
            pltpu.VMEM_SHARED((N_PAD, QW), jnp.float32),
            pltpu.VMEM_SHARED((N_PAD,), jnp.float32),
            pltpu.SemaphoreType.DMA, pltpu.SemaphoreType.DMA,
            pltpu.SemaphoreType.DMA, pltpu.SemaphoreType.DMA,
            pltpu.SemaphoreType.DMA, pltpu.SemaphoreType.DMA,
            pltpu.SemaphoreType.DMA, pltpu.SemaphoreType.DMA,
            pltpu.SemaphoreType.DMA,
        ],
    )


@jax.jit
def kernel(x, edge_index, edge_weight):
    xt = x.reshape(4 * N_NODES, QW)
    src2 = edge_index[0].reshape(NCH_TOT, CHUNK)
    dst2 = edge_index[1].reshape(NCH_TOT, CHUNK)
    w2 = edge_weight.reshape(NCH_TOT, CHUNK)
    o00, o01, o10, o11 = _make_kernel()(xt, src2, dst2, w2)
    return jnp.concatenate([o00, o01, o10, o11], axis=1)
